# baseline (speedup 1.0000x reference)
import dataclasses
import functools

import jax
import jax.numpy as jnp
import numpy as np
from jax import lax
from jax.experimental import pallas as pl
from jax.experimental.pallas import tpu as pltpu
from jax.experimental.pallas import tpu_sc as plsc
from jax.sharding import Mesh, PartitionSpec as P

try:
    from jax import shard_map as _shard_map
except ImportError:
    from jax.experimental.shard_map import shard_map as _shard_map

LANES = 16
ROWS_PER_BLOCK = 16
NUM_WORKERS = 32


def _densify(scat_idx, w_r, num_blocks, block_elems):
    k_steps = scat_idx.shape[0] // (num_blocks * LANES)
    blocks_per_worker = num_blocks // NUM_WORKERS
    mesh = plsc.VectorSubcoreMesh(
        core_axis_name="c", subcore_axis_name="s", num_cores=2,
        num_subcores=16)

    in_f = block_elems // ROWS_PER_BLOCK
    cp = pltpu.CompilerParams()
    if "needs_layout_passes" in pltpu.CompilerParams.__dataclass_fields__:
        cp = dataclasses.replace(cp, needs_layout_passes=False)

    row_elems = k_steps * LANES

    @functools.partial(
        pl.kernel,
        out_type=jax.ShapeDtypeStruct((num_blocks * ROWS_PER_BLOCK, in_f),
                                      jnp.float32),
        mesh=mesh,
        compiler_params=cp,
        scratch_types=[
            pltpu.VMEM((ROWS_PER_BLOCK, in_f), jnp.float32),
            pltpu.VMEM((ROWS_PER_BLOCK, in_f), jnp.float32),
            pltpu.VMEM((blocks_per_worker * row_elems,), jnp.int32),
            pltpu.VMEM((blocks_per_worker * row_elems,), jnp.float32),
            pltpu.SemaphoreType.DMA,
            pltpu.SemaphoreType.DMA,
        ],
    )
    def sc_kernel(scat_hbm, w_hbm, out_hbm, buf0, buf1, idx_v, w_v, s0, s1):
        wid = lax.axis_index("s") * 2 + lax.axis_index("c")
        zeros16 = jnp.zeros((LANES,), jnp.float32)
        lane = lax.iota(jnp.int32, LANES)
        cols = [lane * k_steps + kk for kk in range(k_steps)]
        bufs, sems = (buf0, buf1), (s0, s1)
        base = wid * blocks_per_worker * row_elems

        c_idx = pltpu.async_copy(
            scat_hbm.at[pl.ds(base, blocks_per_worker * row_elems)], idx_v, s0)
        c_w = pltpu.async_copy(
            w_hbm.at[pl.ds(base, blocks_per_worker * row_elems)], w_v, s1)

        for bb in range(2):
            for r in range(ROWS_PER_BLOCK):
                @pl.loop(0, in_f, step=8 * LANES)
                def _(i, bb=bb, r=r):
                    for u in range(8):
                        bufs[bb][r, pl.ds(i + u * LANES, LANES)] = zeros16
        c_idx.wait()
        c_w.wait()

        out_copies = [None, None]
        for j in range(blocks_per_worker):
            buf, sem = bufs[j % 2], sems[j % 2]
            if out_copies[j % 2] is not None:
                out_copies[j % 2].wait()
                for kk in range(k_steps):
                    off = (j - 2) * row_elems
                    iv = plsc.load_gather(idx_v, [cols[kk] + off])
                    plsc.store_scatter(buf, [lane, iv], zeros16)
            for kk in range(k_steps):
                off = j * row_elems
                iv = plsc.load_gather(idx_v, [cols[kk] + off])
                wv = plsc.load_gather(w_v, [cols[kk] + off])
                plsc.addupdate_scatter(buf, [lane, iv], wv)
            b = wid * blocks_per_worker + j
            out_copies[j % 2] = pltpu.async_copy(
                buf, out_hbm.at[pl.ds(b * ROWS_PER_BLOCK, ROWS_PER_BLOCK), :],
                sem)
        for c in out_copies:
            if c is not None:
                c.wait()

    return sc_kernel(scat_idx, w_r)


def _matmul_bias(x2d, wt, bias2d):
    m, kdim = x2d.shape
    o_dim = wt.shape[0]
    bm, bn = 1024, 1024

    def body(x_ref, w_ref, b_ref, o_ref):
        acc = lax.dot_general(
            x_ref[...], w_ref[...].astype(jnp.bfloat16),
            (((1,), (1,)), ((), ())),
            preferred_element_type=jnp.float32,
            precision=lax.Precision.DEFAULT,
        )
        o_ref[...] = acc + b_ref[...]

    return pl.pallas_call(
        body,
        grid=(m // bm, o_dim // bn),
        in_specs=[
            pl.BlockSpec((bm, kdim), lambda i, j: (i, 0)),
            pl.BlockSpec((bn, kdim), lambda i, j: (j, 0)),
            pl.BlockSpec((1, bn), lambda i, j: (0, j)),
        ],
        out_specs=pl.BlockSpec((bm, bn), lambda i, j: (i, j)),
        out_shape=jax.ShapeDtypeStruct((m, o_dim), jnp.float32),
    )(x2d, wt, bias2d)


def _one_device(x2d, idx, weight, bias2d):
    out_f = idx.shape[0]
    in_f = x2d.shape[-1]
    wt = _densify(idx.reshape(-1), weight.reshape(-1),
                  out_f // ROWS_PER_BLOCK, ROWS_PER_BLOCK * in_f)
    return _matmul_bias(x2d, wt, bias2d)


def kernel(x, in_index_per_out, weight, bias):
    out_f, k = in_index_per_out.shape
    in_f = x.shape[-1]
    x2d = x.reshape(-1, in_f).astype(jnp.bfloat16)
    idx = in_index_per_out.astype(jnp.int32)
    bias2d = bias.reshape(1, out_f)

    devs = jax.devices()
    shardable = (len(devs) >= 2
                 and (out_f // ROWS_PER_BLOCK) % (2 * NUM_WORKERS) == 0)
    if shardable:
        mesh = Mesh(np.asarray(devs[:2]), ("d",))
        out = _shard_map(
            _one_device, mesh=mesh,
            in_specs=(P(), P("d"), P("d"), P(None, "d")),
            out_specs=P(None, "d"),
            check_vma=False,
        )(x2d, idx, weight, bias2d)
    else:
        out = _one_device(x2d, idx, weight, bias2d)
    return out.reshape(*x.shape[:-1], out_f)


# speedup vs baseline: 9.6364x; 9.6364x over previous
import dataclasses
import functools

import jax
import jax.numpy as jnp
from jax import lax
from jax.experimental import pallas as pl
from jax.experimental.pallas import tpu as pltpu
from jax.experimental.pallas import tpu_sc as plsc

LANES = 16
ROWS_PER_BLOCK = 16
NUM_WORKERS = 32


def _densify(scat_idx, w_r, num_blocks, block_elems):
    k_steps = scat_idx.shape[0] // (num_blocks * LANES)
    blocks_per_worker = num_blocks // NUM_WORKERS
    mesh = plsc.VectorSubcoreMesh(
        core_axis_name="c", subcore_axis_name="s", num_cores=2,
        num_subcores=16)

    in_f = block_elems // ROWS_PER_BLOCK
    cp = pltpu.CompilerParams()
    if "needs_layout_passes" in pltpu.CompilerParams.__dataclass_fields__:
        cp = dataclasses.replace(cp, needs_layout_passes=False)

    row_elems = k_steps * LANES

    @functools.partial(
        pl.kernel,
        out_type=jax.ShapeDtypeStruct((num_blocks * ROWS_PER_BLOCK, in_f),
                                      jnp.float32),
        mesh=mesh,
        compiler_params=cp,
        scratch_types=[
            pltpu.VMEM((ROWS_PER_BLOCK, in_f), jnp.float32),
            pltpu.VMEM((ROWS_PER_BLOCK, in_f), jnp.float32),
            pltpu.VMEM((blocks_per_worker * row_elems,), jnp.int32),
            pltpu.VMEM((blocks_per_worker * row_elems,), jnp.float32),
            pltpu.SemaphoreType.DMA,
            pltpu.SemaphoreType.DMA,
        ],
    )
    def sc_kernel(scat_hbm, w_hbm, out_hbm, buf0, buf1, idx_v, w_v, s0, s1):
        wid = lax.axis_index("s") * 2 + lax.axis_index("c")
        zeros16 = jnp.zeros((LANES,), jnp.float32)
        lane = lax.iota(jnp.int32, LANES)
        cols = [lane * k_steps + kk for kk in range(k_steps)]
        bufs, sems = (buf0, buf1), (s0, s1)
        base = wid * blocks_per_worker * row_elems

        c_idx = pltpu.async_copy(
            scat_hbm.at[pl.ds(base, blocks_per_worker * row_elems)], idx_v, s0)
        c_w = pltpu.async_copy(
            w_hbm.at[pl.ds(base, blocks_per_worker * row_elems)], w_v, s1)

        for bb in range(2):
            @pl.loop(0, ROWS_PER_BLOCK)
            def _(r, bb=bb):
                @pl.loop(0, in_f, step=8 * LANES)
                def _(i, bb=bb, r=r):
                    for u in range(8):
                        bufs[bb][r, pl.ds(i + u * LANES, LANES)] = zeros16
        c_idx.wait()
        c_w.wait()

        out_copies = [None, None]
        for j in range(blocks_per_worker):
            buf, sem = bufs[j % 2], sems[j % 2]
            if out_copies[j % 2] is not None:
                out_copies[j % 2].wait()
                for kk in range(k_steps):
                    off = (j - 2) * row_elems
                    iv = plsc.load_gather(idx_v, [cols[kk] + off])
                    plsc.store_scatter(buf, [lane, iv], zeros16)
            for kk in range(k_steps):
                off = j * row_elems
                iv = plsc.load_gather(idx_v, [cols[kk] + off])
                wv = plsc.load_gather(w_v, [cols[kk] + off])
                plsc.addupdate_scatter(buf, [lane, iv], wv)
            b = wid * blocks_per_worker + j
            out_copies[j % 2] = pltpu.async_copy(
                buf, out_hbm.at[pl.ds(b * ROWS_PER_BLOCK, ROWS_PER_BLOCK), :],
                sem)
        for c in out_copies:
            if c is not None:
                c.wait()

    return sc_kernel(scat_idx, w_r)


def _matmul_bias(x2d, wt, bias2d):
    m, kdim = x2d.shape
    o_dim = wt.shape[0]
    bm, bn = 1024, 1024

    def body(x_ref, w_ref, b_ref, o_ref):
        acc = lax.dot_general(
            x_ref[...], w_ref[...].astype(jnp.bfloat16),
            (((1,), (1,)), ((), ())),
            preferred_element_type=jnp.float32,
            precision=lax.Precision.DEFAULT,
        )
        o_ref[...] = acc + b_ref[...]

    return pl.pallas_call(
        body,
        grid=(m // bm, o_dim // bn),
        in_specs=[
            pl.BlockSpec((bm, kdim), lambda i, j: (i, 0)),
            pl.BlockSpec((bn, kdim), lambda i, j: (j, 0)),
            pl.BlockSpec((1, bn), lambda i, j: (0, j)),
        ],
        out_specs=pl.BlockSpec((bm, bn), lambda i, j: (i, j)),
        out_shape=jax.ShapeDtypeStruct((m, o_dim), jnp.float32),
    )(x2d, wt, bias2d)


def kernel(x, in_index_per_out, weight, bias):
    out_f, k = in_index_per_out.shape
    in_f = x.shape[-1]
    x2d = x.reshape(-1, in_f)
    num_blocks = out_f // ROWS_PER_BLOCK
    block_elems = ROWS_PER_BLOCK * in_f

    idx_r = in_index_per_out.astype(jnp.int32).reshape(-1)
    w_r = weight.reshape(-1)

    wt = _densify(idx_r, w_r, num_blocks, block_elems)
    out = _matmul_bias(x2d.astype(jnp.bfloat16), wt, bias.reshape(1, out_f))
    return out.reshape(*x.shape[:-1], out_f)
